# baseline (speedup 1.0000x reference)
import functools

import jax
import jax.numpy as jnp
from jax.experimental import pallas as pl
from jax.experimental.pallas import tpu as pltpu

_CH = 32
_UNROLL = 4


def _collapse(ms, desc):
    K = len(ms)
    sel_hi = jnp.maximum if desc else jnp.minimum
    while ms[0].shape[0] > 8:
        h = ms[0].shape[0] // 2
        a = [m[:h] for m in ms]
        b = [m[h:] for m in ms]
        part = [sel_hi(a[j], b[K - 1 - j]) for j in range(K)]
        for d in (4, 2, 1):
            nxt = list(part)
            for j in range(K):
                if j % (2 * d) < d:
                    hi = jnp.maximum(part[j], part[j + d])
                    lo = jnp.minimum(part[j], part[j + d])
                    nxt[j] = hi if desc else lo
                    nxt[j + d] = lo if desc else hi
            part = nxt
        ms = part
    return ms


def _adj_body(K, x_ref, prob_ref, cond_ref, ent_ref, sm_ref):
    V, F = x_ref.shape[1], x_ref.shape[2]
    NCH = V // (_CH * _UNROLL)

    row_iota = jax.lax.broadcasted_iota(jnp.int32, (_CH, F), 0)

    def pass_a(i, carry):
        carry = list(carry)
        for u in range(_UNROLL):
            g = carry[u * (K + 1):(u + 1) * (K + 1)]
            v = x_ref[0, pl.ds((i * _UNROLL + u) * _CH, _CH), :]
            g[0] = g[0] + v
            cur = v
            for j in range(K):
                hi = jnp.maximum(g[1 + j], cur)
                cur = jnp.minimum(g[1 + j], cur)
                g[1 + j] = hi
            carry[u * (K + 1):(u + 1) * (K + 1)] = g
        return tuple(carry)

    init_a = (
        (jnp.zeros((_CH, F), jnp.float32),) + tuple(
            jnp.full((_CH, F), -jnp.inf, jnp.float32) for _ in range(K))
    ) * _UNROLL
    res_a = jax.lax.fori_loop(0, NCH, pass_a, init_a)
    s = sum(jnp.sum(res_a[u * (K + 1)], axis=0, keepdims=True)
            for u in range(_UNROLL))
    rcp = 1.0 / (s + 1e-20)
    ms = [jnp.concatenate([res_a[u * (K + 1) + 1 + j]
                           for u in range(_UNROLL)], axis=0)
          for j in range(K)]
    cand_x = jnp.concatenate(_collapse(ms, True), axis=0)
    sm_cand = jnp.clip(cand_x * rcp, 0.001, 1.0 - 0.001)

    t = jnp.full((1, F), 2.0, jnp.float32)
    n = jnp.zeros((1, F), jnp.int32)
    c = jnp.zeros((1, F), jnp.int32)
    for _ in range(K):
        m = jnp.max(jnp.where(sm_cand < t, sm_cand, -1.0), axis=0,
                    keepdims=True)
        n_new = jnp.sum((sm_cand >= m).astype(jnp.int32), axis=0,
                        keepdims=True)
        upd = n < K
        c = jnp.where(upd, n, c)
        t = jnp.where(upd, m, t)
        n = jnp.where(upd, n_new, n)
    e = K - c

    def pass_b(i, carry):
        carry = list(carry)
        for u in range(_UNROLL):
            g = carry[u * (K + 1):(u + 1) * (K + 1)]
            base = (i * _UNROLL + u) * _CH
            xv = x_ref[0, pl.ds(base, _CH), :]
            smv = jnp.clip(xv * rcp, 0.001, 1.0 - 0.001)
            sm_ref[0, pl.ds(base, _CH), :] = smv
            g[0] = g[0] - smv * jnp.log(smv)
            cur = jnp.where(smv == t, row_iota + base, V)
            for j in range(K):
                lo = jnp.minimum(g[1 + j], cur)
                cur = jnp.maximum(g[1 + j], cur)
                g[1 + j] = lo
            carry[u * (K + 1):(u + 1) * (K + 1)] = g
        return tuple(carry)

    init_b = (
        (jnp.zeros((_CH, F), jnp.float32),) + tuple(
            jnp.full((_CH, F), V, jnp.int32) for _ in range(K))
    ) * _UNROLL
    res_b = jax.lax.fori_loop(0, NCH, pass_b, init_b)
    ent_total = sum(jnp.sum(res_b[u * (K + 1)]) for u in range(_UNROLL))
    ent_ref[...] = (ent_total / F).reshape(1, 1, 1)
    js = [jnp.concatenate([res_b[u * (K + 1) + 1 + j]
                           for u in range(_UNROLL)], axis=0)
          for j in range(K)]
    cand_i = jnp.concatenate(_collapse(js, False), axis=0)

    s_last = jnp.full((1, F), -1, jnp.int32)
    last = jnp.full((1, F), -1, jnp.int32)
    for i in range(K):
        cnd = jnp.min(jnp.where(cand_i > last, cand_i, V), axis=0,
                      keepdims=True)
        s_last = jnp.where(i < e, cnd, s_last)
        last = cnd

    thr = 1.0 / (V * K)

    def pass_f(i, carry):
        for u in range(_UNROLL):
            base = (i * _UNROLL + u) * _CH
            sl = pl.ds(base, _CH)
            smv = sm_ref[0, sl, :]
            rmax = s_last - base
            mask = (smv > t) | ((smv == t) & (row_iota <= rmax))
            mask = mask & (smv > thr)
            cond_ref[0, sl, :] = mask.astype(jnp.int32)
            prob_ref[0, sl, :] = jnp.where(mask, jnp.log(smv), 0.0)
        return carry

    jax.lax.fori_loop(0, NCH, pass_f, 0)


def kernel(stack_exp):
    B, V, F = stack_exp.shape
    K = 8
    slab = pl.BlockSpec((1, V, F), lambda b: (b, 0, 0))
    prob, cond, ent, sm = pl.pallas_call(
        functools.partial(_adj_body, K),
        grid=(B,),
        in_specs=[slab],
        out_specs=[slab, slab,
                   pl.BlockSpec((1, 1, 1), lambda b: (b, 0, 0)), slab],
        out_shape=[
            jax.ShapeDtypeStruct((B, V, F), jnp.float32),
            jax.ShapeDtypeStruct((B, V, F), jnp.int32),
            jax.ShapeDtypeStruct((B, 1, 1), jnp.float32),
            jax.ShapeDtypeStruct((B, V, F), jnp.float32),
        ],
        compiler_params=pltpu.CompilerParams(
            dimension_semantics=("parallel",)),
    )(stack_exp)
    return prob, cond, ent.reshape(B), sm
